# baseline (speedup 1.0000x reference)
import functools

import jax
import jax.numpy as jnp
from jax import lax
from jax.experimental import pallas as pl
from jax.experimental.pallas import tpu as pltpu
from jax.experimental.pallas import tpu_sc as plsc

_V = 1_000_000
_D = 64
_B = 16384 * 20
_NW = 32
_B_PER_W = _B // _NW
_CHUNK = 640
_NCHUNK = _B_PER_W // _CHUNK
_SUB = 128
_NSUB = _CHUNK // _SUB

_mesh = plsc.VectorSubcoreMesh(core_axis_name="c", subcore_axis_name="s")


@functools.partial(
    pl.kernel,
    out_type=jax.ShapeDtypeStruct((_B, _D), jnp.float32),
    mesh=_mesh,
    scratch_types=[
        pltpu.VMEM((_CHUNK,), jnp.int32),
        pltpu.VMEM((_CHUNK, _D), jnp.float32),
        pltpu.SemaphoreType.DMA,
    ],
    compiler_params=pltpu.CompilerParams(use_tc_tiling_on_sc=False),
)
def _embed_gather(x_hbm, w_hbm, out_hbm, idx_v, rows_v, sem):
    wid = lax.axis_index("s") * 2 + lax.axis_index("c")
    base = pl.multiple_of(wid * _B_PER_W, _CHUNK)

    def block(c, _):
        off = pl.multiple_of(base + c * _CHUNK, _CHUNK)
        pltpu.sync_copy(x_hbm.at[pl.ds(off, _CHUNK)], idx_v)
        handles = []
        for j in range(_NSUB):
            h = pltpu.async_copy(
                w_hbm.at[idx_v.at[pl.ds(j * _SUB, _SUB)]],
                rows_v.at[pl.ds(j * _SUB, _SUB), :],
                sem,
            )
            handles.append(h)
        for h in handles:
            h.wait()
        pltpu.sync_copy(rows_v, out_hbm.at[pl.ds(off, _CHUNK)])
        return ()

    lax.fori_loop(0, _NCHUNK, block, ())


def kernel(x, weight):
    flat = _embed_gather(x.reshape(-1), weight)
    return flat.reshape(x.shape + (_D,))


# speedup vs baseline: 1.0182x; 1.0182x over previous
import functools

import jax
import jax.numpy as jnp
from jax import lax
from jax.experimental import pallas as pl
from jax.experimental.pallas import tpu as pltpu
from jax.experimental.pallas import tpu_sc as plsc

_V = 1_000_000
_D = 64
_B = 16384 * 20
_NW = 32
_B_PER_W = _B // _NW
_CHUNK = 256
_NCHUNK = _B_PER_W // _CHUNK
_SUB = 128
_NSUB = _CHUNK // _SUB
_RING = 4
_NSUPER = _NCHUNK // _RING

_mesh = plsc.VectorSubcoreMesh(core_axis_name="c", subcore_axis_name="s")


@functools.partial(
    pl.kernel,
    out_type=jax.ShapeDtypeStruct((_B, _D), jnp.float32),
    mesh=_mesh,
    scratch_types=[
        pltpu.VMEM((_B_PER_W,), jnp.int32),
        pltpu.VMEM((_RING, _CHUNK, _D), jnp.float32),
        pltpu.SemaphoreType.DMA,
        pltpu.SemaphoreType.DMA,
        pltpu.SemaphoreType.DMA,
        pltpu.SemaphoreType.DMA,
        pltpu.SemaphoreType.DMA,
    ],
    compiler_params=pltpu.CompilerParams(use_tc_tiling_on_sc=False),
)
def _embed_gather(x_hbm, w_hbm, out_hbm, idx_v, rows_v, sem_g,
                  sem_w0, sem_w1, sem_w2, sem_w3):
    sem_w = (sem_w0, sem_w1, sem_w2, sem_w3)
    wid = lax.axis_index("s") * 2 + lax.axis_index("c")
    base = pl.multiple_of(wid * _B_PER_W, _B_PER_W)

    pltpu.sync_copy(x_hbm.at[pl.ds(base, _B_PER_W)], idx_v)

    def fire_gathers(loc, p):
        for j in range(_NSUB):
            pltpu.async_copy(
                w_hbm.at[idx_v.at[pl.ds(loc + j * _SUB, _SUB)]],
                rows_v.at[p, pl.ds(j * _SUB, _SUB), :],
                sem_g,
            )

    for p in range(_RING):
        fire_gathers(p * _CHUNK, p)

    def super_block(g, _):
        for p in range(_RING):
            k = g * _RING + p
            loc = pl.multiple_of(k * _CHUNK, _CHUNK)
            pltpu.make_async_copy(
                w_hbm.at[pl.ds(0, _CHUNK)], rows_v.at[p], sem_g
            ).wait()
            pltpu.async_copy(
                rows_v.at[p], out_hbm.at[pl.ds(base + loc, _CHUNK)], sem_w[p]
            )

            @pl.when(g < _NSUPER - 1)
            def _():
                pltpu.make_async_copy(
                    rows_v.at[p], out_hbm.at[pl.ds(base, _CHUNK)], sem_w[p]
                ).wait()
                fire_gathers(loc + _RING * _CHUNK, p)

        return ()

    lax.fori_loop(0, _NSUPER, super_block, ())

    for p in range(_RING):
        pltpu.make_async_copy(
            rows_v.at[p], out_hbm.at[pl.ds(base, _CHUNK)], sem_w[p]
        ).wait()


def kernel(x, weight):
    flat = _embed_gather(x.reshape(-1), weight)
    return flat.reshape(x.shape + (_D,))


# speedup vs baseline: 1.0199x; 1.0016x over previous
import functools

import jax
import jax.numpy as jnp
from jax import lax
from jax.experimental import pallas as pl
from jax.experimental.pallas import tpu as pltpu
from jax.experimental.pallas import tpu_sc as plsc

_V = 1_000_000
_D = 64
_R = 16384
_C = 20
_NW = 32
_ROWS_PER_W = _R // _NW
_B_PER_W = _ROWS_PER_W * _C
_SUB = 128
_BLK_ROWS = 32
_CHUNK = _BLK_ROWS * _C
_NSUB = _CHUNK // _SUB
_NCHUNK = _ROWS_PER_W // _BLK_ROWS
_RING = 2
_NSUPER = _NCHUNK // _RING

_mesh = plsc.VectorSubcoreMesh(core_axis_name="c", subcore_axis_name="s")


@functools.partial(
    pl.kernel,
    out_type=jax.ShapeDtypeStruct((_R, _C, _D), jnp.float32),
    mesh=_mesh,
    scratch_types=[
        pltpu.VMEM((1, _B_PER_W), jnp.int32),
        pltpu.VMEM((_RING * _CHUNK, _D), jnp.float32),
        pltpu.SemaphoreType.DMA,
        pltpu.SemaphoreType.DMA,
        pltpu.SemaphoreType.DMA,
    ],
    compiler_params=pltpu.CompilerParams(use_tc_tiling_on_sc=False),
)
def _embed_gather(x_hbm, w_hbm, out_hbm, idx_v, rows_v, sem_g,
                  sem_w0, sem_w1):
    sem_w = (sem_w0, sem_w1)
    wid = lax.axis_index("s") * 2 + lax.axis_index("c")
    row0 = pl.multiple_of(wid * _ROWS_PER_W, _ROWS_PER_W)

    pltpu.sync_copy(x_hbm.at[pl.ds(wid, 1), :], idx_v)

    def fire_gathers(k, p):
        loc = pl.multiple_of(k * _CHUNK, _CHUNK)
        for j in range(_NSUB):
            pltpu.async_copy(
                w_hbm.at[idx_v.at[0, pl.ds(loc + j * _SUB, _SUB)]],
                rows_v.at[pl.ds(p * _CHUNK + j * _SUB, _SUB), :],
                sem_g,
            )

    for p in range(_RING):
        fire_gathers(p, p)

    def super_block(g, _):
        for p in range(_RING):
            k = g * _RING + p
            blk_row = row0 + k * _BLK_ROWS
            slot = rows_v.at[pl.ds(p * _CHUNK, _CHUNK), :]
            pltpu.make_async_copy(
                w_hbm.at[pl.ds(0, _CHUNK)], slot, sem_g
            ).wait()

            def write_row(r, _):
                pltpu.async_copy(
                    rows_v.at[pl.ds(p * _CHUNK + r * _C, _C), :],
                    out_hbm.at[blk_row + r, :, :],
                    sem_w[p],
                )
                return ()
            lax.fori_loop(0, _BLK_ROWS, write_row, ())

            @pl.when(g < _NSUPER - 1)
            def _():
                def wait_row(r, _):
                    pltpu.make_async_copy(
                        rows_v.at[pl.ds(0, _C), :],
                        out_hbm.at[0, :, :],
                        sem_w[p],
                    ).wait()
                    return ()
                lax.fori_loop(0, _BLK_ROWS, wait_row, ())
                fire_gathers(k + _RING, p)

        return ()

    lax.fori_loop(0, _NSUPER, super_block, ())

    for p in range(_RING):
        def wait_row(r, _):
            pltpu.make_async_copy(
                rows_v.at[pl.ds(0, _C), :],
                out_hbm.at[0, :, :],
                sem_w[p],
            ).wait()
            return ()
        lax.fori_loop(0, _BLK_ROWS, wait_row, ())


def kernel(x, weight):
    return _embed_gather(x.reshape(_NW, _B_PER_W), weight)
